# Initial kernel scaffold; baseline (speedup 1.0000x reference)
import jax
import jax.numpy as jnp
from jax.experimental import pallas as pl


def kernel(X, ref_points, theta_v):
    raise NotImplementedError("write your pallas kernel here")


# speedup vs baseline: 2.9751x; 2.9751x over previous
import functools

import jax
import jax.numpy as jnp
from jax import lax
from jax.experimental import pallas as pl
from jax.experimental.pallas import tpu as pltpu
from jax.experimental.pallas import tpu_sc as plsc


def _partner(x, d):
    n = x.shape[0]
    tail = x.shape[1:]
    y = x.reshape((n // (2 * d), 2, d) + tail)
    y = jnp.concatenate([y[:, 1:2], y[:, 0:1]], axis=1)
    return y.reshape((n,) + tail)


def _stage_mask(iota, k, d):
    bk = (iota >> int(k).bit_length() - 1) & 1
    bd = (iota >> int(d).bit_length() - 1) & 1
    return 1 - (bk ^ bd)


def _f2i(x):
    return lax.bitcast_convert_type(x, jnp.int32)


def _i2f(x):
    return lax.bitcast_convert_type(x, jnp.float32)


def _bitonic_sort_values(x):
    n = x.shape[0]
    iota = lax.broadcasted_iota(jnp.int32, (n, 1), 0)
    k = 2
    while k <= n:
        d = k // 2
        while d >= 1:
            m = _stage_mask(iota, k, d)
            p = _partner(x, d)
            bmn = _f2i(jnp.minimum(x, p))
            bmx = _f2i(jnp.maximum(x, p))
            x = _i2f(bmx + m * (bmn - bmx))
            d //= 2
        k *= 2
    return x


def _bitonic_argsort(keys):
    n = keys.shape[0]
    iota = lax.broadcasted_iota(jnp.int32, (n, 1), 0)
    vals = lax.broadcasted_iota(jnp.int32, keys.shape, 0)
    k = 2
    while k <= n:
        d = k // 2
        while d >= 1:
            m = _stage_mask(iota, k, d)
            pk = _partner(keys, d)
            pv = _partner(vals, d)
            ltk = (pk < keys).astype(jnp.int32)
            eqk = (pk == keys).astype(jnp.int32)
            ltv = (pv < vals).astype(jnp.int32)
            lt = ltk + eqk * ltv
            gt = 1 - lt
            sel = gt + m * (lt - gt)
            bk = _f2i(keys)
            keys = _i2f(bk + sel * (_f2i(pk) - bk))
            vals = vals + sel * (pv - vals)
            d //= 2
        k *= 2
    return keys, vals


def _prep_body(rs_ref, rt_ref, rindt_ref):
    rs = rs_ref[...]
    rt_ref[...] = rs.T
    _, rind = _bitonic_argsort(rs)
    rindt_ref[...] = rind.T


def _prep(rslices, interpret=False):
    M, L = rslices.shape
    return pl.pallas_call(
        _prep_body,
        out_shape=[
            jax.ShapeDtypeStruct((L, M), jnp.float32),
            jax.ShapeDtypeStruct((L, M), jnp.int32),
        ],
        interpret=interpret,
    )(rslices)


def _proj_sort_body(x_ref, w_ref, out_ref):
    x = x_ref[0]
    xs = lax.dot_general(
        x, w_ref[...], (((1,), (1,)), ((), ())),
        preferred_element_type=jnp.float32,
    )
    xs = _bitonic_sort_values(xs)
    out_ref[0] = xs.T


def _proj_sort(X, w, interpret=False):
    B, N, D = X.shape
    L = w.shape[0]
    return pl.pallas_call(
        _proj_sort_body,
        grid=(B,),
        in_specs=[
            pl.BlockSpec((1, N, D), lambda b: (b, 0, 0)),
            pl.BlockSpec((L, D), lambda b: (0, 0)),
        ],
        out_specs=pl.BlockSpec((1, L, N), lambda b: (b, 0, 0)),
        out_shape=jax.ShapeDtypeStruct((B, L, N), jnp.float32),
        interpret=interpret,
    )(X, w)


def _sc_gather_sub(xsT, rt, rindt):
    B, L, N = xsT.shape
    info = plsc.get_sparse_core_info()
    NC, NS = info.num_cores, info.num_subcores
    NW = NC * NS
    LW = L // NW
    mesh = plsc.VectorSubcoreMesh(core_axis_name="c", subcore_axis_name="s")

    @functools.partial(
        pl.kernel,
        out_type=jax.ShapeDtypeStruct((B, L * N), jnp.float32),
        mesh=mesh,
        compiler_params=pltpu.CompilerParams(needs_layout_passes=False),
        scratch_types=[
            pltpu.VMEM((LW * N,), jnp.float32),
            pltpu.VMEM((LW * N,), jnp.float32),
            pltpu.VMEM((LW * N,), jnp.float32),
            pltpu.VMEM((LW * N,), jnp.int32),
        ],
    )
    def k(xsT_hbm, rt_hbm, rindt_hbm, out_hbm, x_v, o_v, r_v, idx_v):
        wid = lax.axis_index("s") * NC + lax.axis_index("c")
        l0 = wid * LW
        pltpu.sync_copy(rt_hbm.at[pl.ds(l0 * N, LW * N)], r_v)
        pltpu.sync_copy(rindt_hbm.at[pl.ds(l0 * N, LW * N)], idx_v)

        def per_b(b, _):
            pltpu.sync_copy(xsT_hbm.at[b, pl.ds(l0 * N, LW * N)], x_v)

            def per_li(li, _):
                base = jnp.full((16,), li * N, jnp.int32)

                def per_j(j, _):
                    off = li * N + j * 16
                    iv = idx_v[pl.ds(off, 16)] + base
                    g = plsc.load_gather(x_v, [iv])
                    o_v[pl.ds(off, 16)] = r_v[pl.ds(off, 16)] - g
                    return 0

                lax.fori_loop(0, N // 16, per_j, 0)
                return 0

            lax.fori_loop(0, LW, per_li, 0)
            pltpu.sync_copy(o_v, out_hbm.at[b, pl.ds(l0 * N, LW * N)])
            return 0

        lax.fori_loop(0, B, per_b, 0)

    return k(
        xsT.reshape(B, L * N), rt.reshape(L * N), rindt.reshape(L * N)
    )


def kernel(X, ref_points, theta_v):
    W = theta_v / (jnp.linalg.norm(theta_v, axis=1, keepdims=True))
    Rslices = jnp.einsum('md,ld->ml', ref_points, W)
    rt, rindt = _prep(Rslices)
    xsT = _proj_sort(X, W)
    return _sc_gather_sub(xsT, rt, rindt)


# speedup vs baseline: 4.2703x; 1.4353x over previous
import functools

import jax
import jax.numpy as jnp
from jax import lax
from jax.experimental import pallas as pl
from jax.experimental.pallas import tpu as pltpu
from jax.experimental.pallas import tpu_sc as plsc


def _partner(x, d):
    n = x.shape[0]
    tail = x.shape[1:]
    y = x.reshape((n // (2 * d), 2, d) + tail)
    y = jnp.concatenate([y[:, 1:2], y[:, 0:1]], axis=1)
    return y.reshape((n,) + tail)


def _stage_mask(iota, k, d):
    bk = (iota >> int(k).bit_length() - 1) & 1
    bd = (iota >> int(d).bit_length() - 1) & 1
    return 1 - (bk ^ bd)


def _f2i(x):
    return lax.bitcast_convert_type(x, jnp.int32)


def _i2f(x):
    return lax.bitcast_convert_type(x, jnp.float32)


def _bisect_stage(u, d, swap):
    sh = u.shape
    n_u, lanes = sh[-2], sh[-1]
    y = u.reshape(sh[:-2] + (n_u // (2 * d), 2, d, lanes))
    a = y[..., 0:1, :, :]
    b = y[..., 1:2, :, :]
    mn = jnp.minimum(a, b)
    mx = jnp.maximum(a, b)
    lo, hi = (mx, mn) if swap else (mn, mx)
    return jnp.concatenate([lo, hi], axis=-3).reshape(sh)


def _bitonic_sort_values(x):
    n, lanes = x.shape
    k = 2
    while k < n:
        d = k // 2
        while d >= 1:
            v = x.reshape(n // (2 * k), 2, k, lanes)
            xa = _bisect_stage(v[:, 0], d, False)
            xd = _bisect_stage(v[:, 1], d, True)
            x = jnp.concatenate(
                [xa[:, None], xd[:, None]], axis=1
            ).reshape(n, lanes)
            d //= 2
        k *= 2
    d = n // 2
    while d >= 1:
        x = _bisect_stage(x, d, False)
        d //= 2
    return x


def _bitonic_argsort(keys):
    n = keys.shape[0]
    iota = lax.broadcasted_iota(jnp.int32, (n, 1), 0)
    vals = lax.broadcasted_iota(jnp.int32, keys.shape, 0)
    k = 2
    while k <= n:
        d = k // 2
        while d >= 1:
            m = _stage_mask(iota, k, d)
            pk = _partner(keys, d)
            pv = _partner(vals, d)
            ltk = (pk < keys).astype(jnp.int32)
            eqk = (pk == keys).astype(jnp.int32)
            ltv = (pv < vals).astype(jnp.int32)
            lt = ltk + eqk * ltv
            gt = 1 - lt
            sel = gt + m * (lt - gt)
            bk = _f2i(keys)
            keys = _i2f(bk + sel * (_f2i(pk) - bk))
            vals = vals + sel * (pv - vals)
            d //= 2
        k *= 2
    return keys, vals


def _prep_body(rs_ref, rt_ref, rindt_ref):
    rs = rs_ref[...]
    rt_ref[...] = rs.T
    _, rind = _bitonic_argsort(rs)
    rindt_ref[...] = rind.T


def _prep(rslices, interpret=False):
    M, L = rslices.shape
    return pl.pallas_call(
        _prep_body,
        out_shape=[
            jax.ShapeDtypeStruct((L, M), jnp.float32),
            jax.ShapeDtypeStruct((L, M), jnp.int32),
        ],
        interpret=interpret,
    )(rslices)


def _proj_sort_body(x_ref, w_ref, out_ref):
    x = x_ref[0]
    xs = lax.dot_general(
        x, w_ref[...], (((1,), (1,)), ((), ())),
        preferred_element_type=jnp.float32,
    )
    xs = _bitonic_sort_values(xs)
    out_ref[0] = xs.T


def _proj_sort(X, w, interpret=False):
    B, N, D = X.shape
    L = w.shape[0]
    return pl.pallas_call(
        _proj_sort_body,
        grid=(B,),
        in_specs=[
            pl.BlockSpec((1, N, D), lambda b: (b, 0, 0)),
            pl.BlockSpec((L, D), lambda b: (0, 0)),
        ],
        out_specs=pl.BlockSpec((1, L, N), lambda b: (b, 0, 0)),
        out_shape=jax.ShapeDtypeStruct((B, L, N), jnp.float32),
        interpret=interpret,
    )(X, w)


def _sc_gather_sub(xsT, rt, rindt):
    B, L, N = xsT.shape
    info = plsc.get_sparse_core_info()
    NC, NS = info.num_cores, info.num_subcores
    NW = NC * NS
    LW = L // NW
    mesh = plsc.VectorSubcoreMesh(core_axis_name="c", subcore_axis_name="s")

    @functools.partial(
        pl.kernel,
        out_type=jax.ShapeDtypeStruct((B, L * N), jnp.float32),
        mesh=mesh,
        compiler_params=pltpu.CompilerParams(needs_layout_passes=False),
        scratch_types=[
            pltpu.VMEM((LW * N,), jnp.float32),
            pltpu.VMEM((LW * N,), jnp.float32),
            pltpu.VMEM((LW * N,), jnp.float32),
            pltpu.VMEM((LW * N,), jnp.int32),
        ],
    )
    def k(xsT_hbm, rt_hbm, rindt_hbm, out_hbm, x_v, o_v, r_v, idx_v):
        wid = lax.axis_index("s") * NC + lax.axis_index("c")
        l0 = wid * LW
        pltpu.sync_copy(rt_hbm.at[pl.ds(l0 * N, LW * N)], r_v)
        pltpu.sync_copy(rindt_hbm.at[pl.ds(l0 * N, LW * N)], idx_v)

        def per_b(b, _):
            pltpu.sync_copy(xsT_hbm.at[b, pl.ds(l0 * N, LW * N)], x_v)

            def per_li(li, _):
                base = jnp.full((16,), li * N, jnp.int32)

                def per_j(j, _):
                    off = li * N + j * 16
                    iv = idx_v[pl.ds(off, 16)] + base
                    g = plsc.load_gather(x_v, [iv])
                    o_v[pl.ds(off, 16)] = r_v[pl.ds(off, 16)] - g
                    return 0

                lax.fori_loop(0, N // 16, per_j, 0)
                return 0

            lax.fori_loop(0, LW, per_li, 0)
            pltpu.sync_copy(o_v, out_hbm.at[b, pl.ds(l0 * N, LW * N)])
            return 0

        lax.fori_loop(0, B, per_b, 0)

    return k(
        xsT.reshape(B, L * N), rt.reshape(L * N), rindt.reshape(L * N)
    )


def kernel(X, ref_points, theta_v):
    W = theta_v / (jnp.linalg.norm(theta_v, axis=1, keepdims=True))
    Rslices = jnp.einsum('md,ld->ml', ref_points, W)
    rt, rindt = _prep(Rslices)
    xsT = _proj_sort(X, W)
    return _sc_gather_sub(xsT, rt, rindt)


# speedup vs baseline: 5.1972x; 1.2171x over previous
import functools

import jax
import jax.numpy as jnp
from jax import lax
from jax.experimental import pallas as pl
from jax.experimental.pallas import tpu as pltpu
from jax.experimental.pallas import tpu_sc as plsc


def _partner(x, d):
    n = x.shape[0]
    tail = x.shape[1:]
    y = x.reshape((n // (2 * d), 2, d) + tail)
    y = jnp.concatenate([y[:, 1:2], y[:, 0:1]], axis=1)
    return y.reshape((n,) + tail)


def _stage_mask(iota, k, d):
    bk = (iota >> int(k).bit_length() - 1) & 1
    bd = (iota >> int(d).bit_length() - 1) & 1
    return 1 - (bk ^ bd)


def _f2i(x):
    return lax.bitcast_convert_type(x, jnp.int32)


def _i2f(x):
    return lax.bitcast_convert_type(x, jnp.float32)


def _bisect_stage(u, d, swap):
    sh = u.shape
    n_u, lanes = sh[-2], sh[-1]
    y = u.reshape(sh[:-2] + (n_u // (2 * d), 2, d, lanes))
    a = y[..., 0:1, :, :]
    b = y[..., 1:2, :, :]
    mn = jnp.minimum(a, b)
    mx = jnp.maximum(a, b)
    lo, hi = (mx, mn) if swap else (mn, mx)
    return jnp.concatenate([lo, hi], axis=-3).reshape(sh)


def _bitonic_sort_values(x):
    n, lanes = x.shape
    k = 2
    while k < n:
        d = k // 2
        while d >= 1:
            v = x.reshape(n // (2 * k), 2, k, lanes)
            xa = _bisect_stage(v[:, 0], d, False)
            xd = _bisect_stage(v[:, 1], d, True)
            x = jnp.concatenate(
                [xa[:, None], xd[:, None]], axis=1
            ).reshape(n, lanes)
            d //= 2
        k *= 2
    d = n // 2
    while d >= 1:
        x = _bisect_stage(x, d, False)
        d //= 2
    return x


def _bitonic_argsort(keys):
    n = keys.shape[0]
    iota = lax.broadcasted_iota(jnp.int32, (n, 1), 0)
    vals = lax.broadcasted_iota(jnp.int32, keys.shape, 0)
    k = 2
    while k <= n:
        d = k // 2
        while d >= 1:
            m = _stage_mask(iota, k, d)
            pk = _partner(keys, d)
            pv = _partner(vals, d)
            ltk = (pk < keys).astype(jnp.int32)
            eqk = (pk == keys).astype(jnp.int32)
            ltv = (pv < vals).astype(jnp.int32)
            lt = ltk + eqk * ltv
            gt = 1 - lt
            sel = gt + m * (lt - gt)
            bk = _f2i(keys)
            keys = _i2f(bk + sel * (_f2i(pk) - bk))
            vals = vals + sel * (pv - vals)
            d //= 2
        k *= 2
    return keys, vals


def _prep_body(rs_ref, rt_ref, rindt_ref):
    rs = rs_ref[...]
    rt_ref[...] = rs.T
    _, rind = _bitonic_argsort(rs)
    m = rs.shape[0]
    lw = rs.shape[1] // 32
    io_l = lax.broadcasted_iota(jnp.int32, (rs.shape[1], 1), 0)
    rindt_ref[...] = rind.T + (io_l % lw) * m


def _prep(rslices, interpret=False):
    M, L = rslices.shape
    return pl.pallas_call(
        _prep_body,
        out_shape=[
            jax.ShapeDtypeStruct((L, M), jnp.float32),
            jax.ShapeDtypeStruct((L, M), jnp.int32),
        ],
        interpret=interpret,
    )(rslices)


def _proj_sort_body(x_ref, w_ref, out_ref):
    x = x_ref[0]
    xs = lax.dot_general(
        x, w_ref[...], (((1,), (1,)), ((), ())),
        preferred_element_type=jnp.float32,
    )
    xs = _bitonic_sort_values(xs)
    out_ref[0] = xs.T


def _proj_sort(X, w, interpret=False):
    B, N, D = X.shape
    L = w.shape[0]
    return pl.pallas_call(
        _proj_sort_body,
        grid=(B,),
        in_specs=[
            pl.BlockSpec((1, N, D), lambda b: (b, 0, 0)),
            pl.BlockSpec((L, D), lambda b: (0, 0)),
        ],
        out_specs=pl.BlockSpec((1, L, N), lambda b: (b, 0, 0)),
        out_shape=jax.ShapeDtypeStruct((B, L, N), jnp.float32),
        interpret=interpret,
    )(X, w)


def _sc_gather_sub(xsT, rt, rindt):
    B, L, N = xsT.shape
    info = plsc.get_sparse_core_info()
    NC, NS = info.num_cores, info.num_subcores
    NW = NC * NS
    LW = L // NW
    mesh = plsc.VectorSubcoreMesh(core_axis_name="c", subcore_axis_name="s")

    W16 = LW * N // 16
    UNROLL = 8

    @functools.partial(
        pl.kernel,
        out_type=jax.ShapeDtypeStruct((B, L * N), jnp.float32),
        mesh=mesh,
        compiler_params=pltpu.CompilerParams(needs_layout_passes=False),
        scratch_types=[
            pltpu.VMEM((LW * N,), jnp.float32),
            pltpu.VMEM((LW * N,), jnp.float32),
            pltpu.VMEM((LW * N,), jnp.float32),
            pltpu.VMEM((LW * N,), jnp.float32),
            pltpu.VMEM((LW * N,), jnp.float32),
            pltpu.VMEM((LW * N,), jnp.int32),
            pltpu.SemaphoreType.DMA,
            pltpu.SemaphoreType.DMA,
            pltpu.SemaphoreType.DMA,
            pltpu.SemaphoreType.DMA,
        ],
    )
    def k(xsT_hbm, rt_hbm, rindt_hbm, out_hbm,
          x0, x1, o0, o1, r_v, idx_v, si0, si1, so0, so1):
        wid = lax.axis_index("s") * NC + lax.axis_index("c")
        l0 = wid * LW
        sl = pl.ds(l0 * N, LW * N)
        pltpu.sync_copy(rt_hbm.at[sl], r_v)
        pltpu.sync_copy(rindt_hbm.at[sl], idx_v)

        def compute(x_v, o_v):
            def per_j(j, _):
                for u in range(UNROLL):
                    off = (j * UNROLL + u) * 16
                    g = plsc.load_gather(x_v, [idx_v[pl.ds(off, 16)]])
                    o_v[pl.ds(off, 16)] = r_v[pl.ds(off, 16)] - g
                return 0

            lax.fori_loop(0, W16 // UNROLL, per_j, 0)

        bufs = ((x0, o0, si0, so0), (x1, o1, si1, so1))
        pltpu.async_copy(xsT_hbm.at[0, sl], x0, si0)

        def step(g, _):
            for ph in range(2):
                x_v, o_v, si, so = bufs[ph]
                xn, _, sin, _ = bufs[1 - ph]
                b = g * 2 + ph
                pltpu.make_async_copy(xsT_hbm.at[b, sl], x_v, si).wait()

                @pl.when(b + 1 < B)
                def _():
                    pltpu.async_copy(xsT_hbm.at[b + 1, sl], xn, sin)

                @pl.when(g > 0)
                def _():
                    pltpu.make_async_copy(o_v, out_hbm.at[b, sl], so).wait()

                compute(x_v, o_v)
                pltpu.async_copy(o_v, out_hbm.at[b, sl], so)
            return 0

        lax.fori_loop(0, B // 2, step, 0)
        pltpu.make_async_copy(o0, out_hbm.at[0, sl], so0).wait()
        pltpu.make_async_copy(o1, out_hbm.at[0, sl], so1).wait()

    return k(
        xsT.reshape(B, L * N), rt.reshape(L * N), rindt.reshape(L * N)
    )


def kernel(X, ref_points, theta_v):
    W = theta_v / (jnp.linalg.norm(theta_v, axis=1, keepdims=True))
    Rslices = jnp.einsum('md,ld->ml', ref_points, W)
    rt, rindt = _prep(Rslices)
    xsT = _proj_sort(X, W)
    return _sc_gather_sub(xsT, rt, rindt)
